# baseline (speedup 1.0000x reference)
import functools

import jax
import jax.numpy as jnp
from jax import lax
from jax.experimental import pallas as pl
from jax.experimental.pallas import tpu as pltpu
from jax.experimental.pallas import tpu_sc as plsc

H = 256
W = 512
S = 64
NB = 64
NCH = 3
UNITS = NB * NCH
PR = 72
UNROLL = 6

_NUM_CORES = 2
_NUM_SUBCORES = 16
NW = _NUM_CORES * _NUM_SUBCORES
UPW = UNITS // NW


def _mesh():
    return plsc.VectorSubcoreMesh(core_axis_name="c", subcore_axis_name="s")


@functools.partial(
    pl.kernel,
    out_type=jax.ShapeDtypeStruct((UNITS * S, S), jnp.float32),
    mesh=_mesh(),
    scratch_types=[
        pltpu.VMEM((NB * 2,), jnp.float32),
        pltpu.VMEM((PR + 8,), jnp.int32),
        pltpu.VMEM((PR + 8,), jnp.int32),
        pltpu.VMEM((PR, W), jnp.float32),
        pltpu.VMEM((PR, W), jnp.float32),
        pltpu.VMEM((S, S), jnp.float32),
        pltpu.VMEM((S,), jnp.int32),
        pltpu.VMEM((S,), jnp.float32),
        pltpu.SemaphoreType.DMA,
        pltpu.SemaphoreType.DMA,
    ],
    compiler_params=pltpu.CompilerParams(needs_layout_passes=False),
)
def _region_sc(img_hbm, gaze_hbm, out_hbm,
               gaze_v, idx0_v, idx1_v, patch0_v, patch1_v, out_v,
               ytabi_v, ytabf_v, sem0, sem1):
    cid = lax.axis_index("c")
    sid = lax.axis_index("s")
    wid = sid * _NUM_CORES + cid

    pltpu.sync_copy(gaze_hbm, gaze_v)

    lane = lax.iota(jnp.int32, 16)
    lane_f = lane.astype(jnp.float32)
    idx_bufs = [idx0_v, idx1_v]
    patch_bufs = [patch0_v, patch1_v]
    sems = [sem0, sem1]

    def unit_gaze(t):
        unit = wid * UPW + t
        b = unit // NCH
        bs = jnp.full((16,), 2 * b, jnp.int32)
        gy = plsc.load_gather(gaze_v, [bs])
        gx = plsc.load_gather(gaze_v, [bs + 1])
        return unit, gy, gx

    def start_gather(t):
        unit, gy, _ = unit_gaze(t)
        yc0 = jnp.clip(gy * H - 32.0, 0.0, H - 1.0)
        yn0 = yc0 / H * 2.0 - 1.0
        ys = ((yn0 + 1.0) * 0.5 * (H - 1)).astype(jnp.int32)
        ubase = jnp.full((16,), unit * H, jnp.int32)
        idx_v = idx_bufs[t % 2]
        for m in range(5):
            rows = jnp.minimum(ys + (lane + 16 * m), H - 1) + ubase
            idx_v[pl.ds(16 * m, 16)] = rows
        return pltpu.async_copy(img_hbm.at[idx_v.at[pl.ds(0, PR)]],
                                patch_bufs[t % 2], sems[t % 2])

    pending = start_gather(0)
    for t in range(UPW):
        nxt = start_gather(t + 1) if t + 1 < UPW else None
        pending.wait()
        pending = nxt

        unit, gy, gx = unit_gaze(t)
        patch_v = patch_bufs[t % 2]

        x0s, x1s, wx0s, wx1s = [], [], [], []
        for jv in range(4):
            xc = jnp.clip(gx * W - 32.0 + (lane_f + 16.0 * jv), 0.0, W - 1.0)
            xn = xc / W * 2.0 - 1.0
            x = (xn + 1.0) * 0.5 * (W - 1)
            x0i = x.astype(jnp.int32)
            wx1 = x - x0i.astype(jnp.float32)
            x0s.append(x0i)
            x1s.append(x0i + 1)
            wx1s.append(wx1)
            wx0s.append(1.0 - wx1)

        yc0 = jnp.clip(gy * H - 32.0, 0.0, H - 1.0)
        yn0 = yc0 / H * 2.0 - 1.0
        ys = ((yn0 + 1.0) * 0.5 * (H - 1)).astype(jnp.int32)

        for m in range(4):
            i_f = lane_f + 16.0 * m
            yc = jnp.clip(gy * H - 32.0 + i_f, 0.0, H - 1.0)
            yn = yc / H * 2.0 - 1.0
            y = (yn + 1.0) * 0.5 * (H - 1)
            y0i = y.astype(jnp.int32)
            ytabi_v[pl.ds(16 * m, 16)] = y0i - ys
            ytabf_v[pl.ds(16 * m, 16)] = y - y0i.astype(jnp.float32)

        @plsc.parallel_loop(0, S, step=1, unroll=UNROLL)
        def row_body(i):
            iv = jnp.full((16,), i, jnp.int32)
            r0 = plsc.load_gather(ytabi_v, [iv])
            wy1 = plsc.load_gather(ytabf_v, [iv])
            wy0 = 1.0 - wy1
            r1 = r0 + 1
            for jv in range(4):
                g00 = plsc.load_gather(patch_v, [r0, x0s[jv]])
                g01 = plsc.load_gather(patch_v, [r0, x1s[jv]])
                g10 = plsc.load_gather(patch_v, [r1, x0s[jv]])
                g11 = plsc.load_gather(patch_v, [r1, x1s[jv]])
                val = (wy0 * (g00 * wx0s[jv] + g01 * wx1s[jv])
                       + wy1 * (g10 * wx0s[jv] + g11 * wx1s[jv]))
                out_v[i, pl.ds(jv * 16, 16)] = val

        pltpu.sync_copy(out_v, out_hbm.at[pl.ds(unit * S, S), :])


def kernel(images, gaze_points):
    img2d = images.reshape(NB * NCH * H, W)
    out = _region_sc(img2d, gaze_points.reshape(NB * 2))
    return out.reshape(NB, NCH, S, S)


# speedup vs baseline: 1.1518x; 1.1518x over previous
import functools

import jax
import jax.numpy as jnp
from jax import lax
from jax.experimental import pallas as pl
from jax.experimental.pallas import tpu as pltpu
from jax.experimental.pallas import tpu_sc as plsc

H = 256
W = 512
S = 64
NB = 64
NCH = 3
UNITS = NB * NCH
PR = 72
UNROLL = 4

_NUM_CORES = 2
_NUM_SUBCORES = 16
NW = _NUM_CORES * _NUM_SUBCORES
UPW = UNITS // NW


def _mesh():
    return plsc.VectorSubcoreMesh(core_axis_name="c", subcore_axis_name="s")


@functools.partial(
    pl.kernel,
    out_type=jax.ShapeDtypeStruct((UNITS * S, S), jnp.float32),
    mesh=_mesh(),
    scratch_types=[
        pltpu.VMEM((NB * 2,), jnp.float32),
        pltpu.VMEM((PR + 8,), jnp.int32),
        pltpu.VMEM((PR + 8,), jnp.int32),
        pltpu.VMEM((PR, W), jnp.float32),
        pltpu.VMEM((PR, W), jnp.float32),
        pltpu.VMEM((S, S), jnp.float32),
        pltpu.VMEM((S,), jnp.int32),
        pltpu.VMEM((S,), jnp.float32),
        pltpu.SemaphoreType.DMA,
        pltpu.SemaphoreType.DMA,
    ],
    compiler_params=pltpu.CompilerParams(needs_layout_passes=False),
)
def _region_sc(img_hbm, gaze_hbm, out_hbm,
               gaze_v, idx0_v, idx1_v, patch0_v, patch1_v, out_v,
               ytabi_v, ytabf_v, sem0, sem1):
    cid = lax.axis_index("c")
    sid = lax.axis_index("s")
    wid = sid * _NUM_CORES + cid

    pltpu.sync_copy(gaze_hbm, gaze_v)

    lane = lax.iota(jnp.int32, 16)
    lane_f = lane.astype(jnp.float32)
    idx_bufs = [idx0_v, idx1_v]
    patch_bufs = [patch0_v, patch1_v]
    sems = [sem0, sem1]

    def unit_gaze(t):
        unit = wid * UPW + t
        b = unit // NCH
        bs = jnp.full((16,), 2 * b, jnp.int32)
        gy = plsc.load_gather(gaze_v, [bs])
        gx = plsc.load_gather(gaze_v, [bs + 1])
        return unit, gy, gx

    def start_gather(t):
        unit, gy, _ = unit_gaze(t)
        yc0 = jnp.clip(gy * H - 32.0, 0.0, H - 1.0)
        yn0 = yc0 / H * 2.0 - 1.0
        ys = ((yn0 + 1.0) * 0.5 * (H - 1)).astype(jnp.int32)
        ubase = jnp.full((16,), unit * H, jnp.int32)
        idx_v = idx_bufs[t % 2]
        for m in range(5):
            rows = jnp.minimum(ys + (lane + 16 * m), H - 1) + ubase
            idx_v[pl.ds(16 * m, 16)] = rows
        return pltpu.async_copy(img_hbm.at[idx_v.at[pl.ds(0, PR)]],
                                patch_bufs[t % 2], sems[t % 2])

    pending = start_gather(0)
    for t in range(UPW):
        nxt = start_gather(t + 1) if t + 1 < UPW else None
        pending.wait()
        pending = nxt

        unit, gy, gx = unit_gaze(t)
        patch_v = patch_bufs[t % 2]

        x0s, x1s, wx0s, wx1s = [], [], [], []
        for jv in range(4):
            xc = jnp.clip(gx * W - 32.0 + (lane_f + 16.0 * jv), 0.0, W - 1.0)
            xn = xc / W * 2.0 - 1.0
            x = (xn + 1.0) * 0.5 * (W - 1)
            x0i = x.astype(jnp.int32)
            wx1 = x - x0i.astype(jnp.float32)
            x0s.append(x0i)
            x1s.append(x0i + 1)
            wx1s.append(wx1)
            wx0s.append(1.0 - wx1)

        yc0 = jnp.clip(gy * H - 32.0, 0.0, H - 1.0)
        yn0 = yc0 / H * 2.0 - 1.0
        ys = ((yn0 + 1.0) * 0.5 * (H - 1)).astype(jnp.int32)

        for m in range(4):
            i_f = lane_f + 16.0 * m
            yc = jnp.clip(gy * H - 32.0 + i_f, 0.0, H - 1.0)
            yn = yc / H * 2.0 - 1.0
            y = (yn + 1.0) * 0.5 * (H - 1)
            y0i = y.astype(jnp.int32)
            ytabi_v[pl.ds(16 * m, 16)] = y0i - ys
            ytabf_v[pl.ds(16 * m, 16)] = y - y0i.astype(jnp.float32)

        @plsc.parallel_loop(0, S, step=1, unroll=UNROLL)
        def row_body(i):
            iv = jnp.full((16,), i, jnp.int32)
            r0 = plsc.load_gather(ytabi_v, [iv])
            wy1 = plsc.load_gather(ytabf_v, [iv])
            wy0 = 1.0 - wy1
            r1 = r0 + 1
            for jv in range(4):
                g00 = plsc.load_gather(patch_v, [r0, x0s[jv]])
                g01 = plsc.load_gather(patch_v, [r0, x1s[jv]])
                g10 = plsc.load_gather(patch_v, [r1, x0s[jv]])
                g11 = plsc.load_gather(patch_v, [r1, x1s[jv]])
                val = (wy0 * (g00 * wx0s[jv] + g01 * wx1s[jv])
                       + wy1 * (g10 * wx0s[jv] + g11 * wx1s[jv]))
                out_v[i, pl.ds(jv * 16, 16)] = val

        pltpu.sync_copy(out_v, out_hbm.at[pl.ds(unit * S, S), :])


def kernel(images, gaze_points):
    img2d = images.reshape(NB * NCH * H, W)
    out = _region_sc(img2d, gaze_points.reshape(NB * 2))
    return out.reshape(NB, NCH, S, S)
